# baseline (speedup 1.0000x reference)
import jax
import jax.numpy as jnp
from jax import lax
from jax.experimental import pallas as pl
from jax.experimental.pallas import tpu as pltpu
from jax.experimental.pallas import tpu_sc as plsc

_NSUB = 16
_ET = 80


def _sc_pad(n):
    return -(-n // (8 * _NSUB)) * (8 * _NSUB)


def _sc_agg_call(hs, src, dst):
    n, fw = hs.shape
    e = src.shape[0]
    npad = _sc_pad(n)
    if npad != n:
        hs = jnp.pad(hs, ((0, npad - n), (0, 0)))
    epc = e // (2 * _NSUB)
    rpc = npad // _NSUB
    assert epc * 2 * _NSUB == e and epc % _ET == 0 and fw == 128

    mesh = plsc.VectorSubcoreMesh(core_axis_name="c", subcore_axis_name="s")

    def body(hs_h, src_h, dst_h, out_h, acc, idx_s, idx_d, rows, sem):
        c = lax.axis_index("c")
        s = lax.axis_index("s")
        e0 = (c * _NSUB + s) * epc
        r0 = s * rpc

        pltpu.sync_copy(hs_h.at[pl.ds(r0, rpc)], acc.at[pl.ds(r0, rpc)])
        plsc.subcore_barrier()

        @pl.loop(0, epc, step=_ET)
        def _(t):
            base = e0 + t
            pltpu.sync_copy(src_h.at[pl.ds(base, _ET)], idx_s)
            pltpu.sync_copy(dst_h.at[pl.ds(base, _ET)], idx_d)
            pltpu.async_copy(hs_h.at[idx_s], rows, sem).wait()
            pltpu.sync_copy(rows, acc.at[idx_d], add=True)

        plsc.subcore_barrier()
        pltpu.sync_copy(acc.at[pl.ds(r0, rpc)], out_h.at[c, pl.ds(r0, rpc)])

    f = pl.kernel(
        body,
        out_type=jax.ShapeDtypeStruct((2, npad, fw), jnp.float32),
        mesh=mesh,
        scratch_types=[
            pltpu.VMEM_SHARED((npad, fw), jnp.float32),
            pltpu.VMEM((_ET,), jnp.int32),
            pltpu.VMEM((_ET,), jnp.int32),
            pltpu.VMEM((_ET, fw), jnp.float32),
            pltpu.SemaphoreType.DMA,
        ],
    )
    out = f(hs, src, dst)
    return out[0, :n] + out[1, :n] - hs[:n]


def _sc_deg_call(dst, n):
    e = dst.shape[0]
    npad = _sc_pad(n)
    fw = 128
    epc = e // (2 * _NSUB)
    rpc = npad // _NSUB
    assert epc * 2 * _NSUB == e and epc % _ET == 0
    ones_n = jnp.ones((npad, fw), jnp.float32)

    mesh = plsc.VectorSubcoreMesh(core_axis_name="c", subcore_axis_name="s")

    def body(dst_h, ones_h, out_h, acc, ones_v, idx_d, sem):
        c = lax.axis_index("c")
        s = lax.axis_index("s")
        e0 = (c * _NSUB + s) * epc
        r0 = s * rpc

        pltpu.sync_copy(ones_h.at[pl.ds(r0, rpc)], acc.at[pl.ds(r0, rpc)])
        pltpu.sync_copy(ones_h.at[pl.ds(0, _ET)], ones_v)
        plsc.subcore_barrier()

        @pl.loop(0, epc, step=_ET)
        def _(t):
            pltpu.sync_copy(dst_h.at[pl.ds(e0 + t, _ET)], idx_d)
            pltpu.sync_copy(ones_v, acc.at[idx_d], add=True)

        plsc.subcore_barrier()
        pltpu.sync_copy(acc.at[pl.ds(r0, rpc)], out_h.at[c, pl.ds(r0, rpc)])

    f = pl.kernel(
        body,
        out_type=jax.ShapeDtypeStruct((2, npad, fw), jnp.float32),
        mesh=mesh,
        scratch_types=[
            pltpu.VMEM_SHARED((npad, fw), jnp.float32),
            pltpu.VMEM((_ET, fw), jnp.float32),
            pltpu.VMEM((_ET,), jnp.int32),
            pltpu.SemaphoreType.DMA,
        ],
    )
    out = f(dst, ones_n)
    return out[0, :n, 0] + out[1, :n, 0] - 1.0


def _mm_bias_kernel(x_ref, w_ref, b_ref, o_ref):
    o_ref[...] = (
        jnp.dot(x_ref[...], w_ref[...], preferred_element_type=jnp.float32)
        + b_ref[...]
    )


def _mm(x, w, b, block=1000):
    n, k = x.shape
    f = w.shape[1]
    b2 = b.reshape(1, f)
    return pl.pallas_call(
        _mm_bias_kernel,
        grid=(n // block,),
        in_specs=[
            pl.BlockSpec((block, k), lambda i: (i, 0)),
            pl.BlockSpec((k, f), lambda i: (0, 0)),
            pl.BlockSpec((1, f), lambda i: (0, 0)),
        ],
        out_specs=pl.BlockSpec((block, f), lambda i: (i, 0)),
        out_shape=jax.ShapeDtypeStruct((n, f), jnp.float32),
    )(x, w, b2)


def _bn(h, g, b, eps=1e-5):
    m = jnp.mean(h, axis=0)
    v = jnp.var(h, axis=0)
    return g * (h - m) * jax.lax.rsqrt(v + eps) + b


def kernel(x, edge_index, batch, params):
    n = x.shape[0]
    src, dst = edge_index[0], edge_index[1]

    deg = _sc_deg_call(dst, n)
    dinv = jax.lax.rsqrt(deg)

    def gcn(h, W, b):
        hs = (h @ W) * dinv[:, None]
        fh = hs.shape[1]
        if fh == 128:
            agg = _sc_agg_call(hs, src, dst)
        else:
            aggs = [_sc_agg_call(hs[:, i:i + 128], src, dst)
                    for i in range(0, fh, 128)]
            agg = jnp.concatenate(aggs, axis=1)
        return dinv[:, None] * agg + b

    def block(h, p):
        res = h @ p['Ws'] + p['bs']
        u = jax.nn.relu(_bn(gcn(h, p['W1'], p['b1']), p['g1'], p['bb1']))
        u = _bn(gcn(u, p['W2'], p['b2']), p['g2'], p['bb2'])
        return jax.nn.relu(u + res)

    h = block(x, params['blk1'])
    h = block(h, params['blk2'])
    nrm = jnp.maximum(jnp.linalg.norm(h, axis=1, keepdims=True), 1e-12)
    z = h / nrm

    zw = z @ params['We']
    edge_probs = jax.nn.sigmoid(jnp.sum(zw[src] * z[dst], axis=-1))

    t = jax.nn.relu(_mm(z, params['Wf1'], params['bf1']))
    hdec = _bn(t, params['gf'], params['bbf'])
    x_recon = _mm(hdec, params['Wf2'], params['bf2'])

    z_g = jax.ops.segment_max(z, batch, num_segments=100)
    z_g_mlp = jax.nn.relu(z_g @ params['Wp1'] + params['bp1']) @ params['Wp2'] + params['bp2']
    return (z, x_recon, edge_probs, z_g, z_g_mlp)


# speedup vs baseline: 1.4230x; 1.4230x over previous
import jax
import jax.numpy as jnp
from jax import lax
from jax.experimental import pallas as pl
from jax.experimental.pallas import tpu as pltpu
from jax.experimental.pallas import tpu_sc as plsc

_NSUB = 16
_ET = 128


def _sc_pad(n):
    return -(-n // (8 * _NSUB)) * (8 * _NSUB)


def _sc_agg_call(hs, src, dst):
    n, fw = hs.shape
    e = src.shape[0]
    npad = _sc_pad(n)
    if npad != n:
        hs = jnp.pad(hs, ((0, npad - n), (0, 0)))
    epc = e // (2 * _NSUB)
    rpc = npad // _NSUB
    nt = epc // _ET
    tail = epc - nt * _ET
    assert epc * 2 * _NSUB == e and fw == 128
    assert nt % 2 == 0 and tail % 8 == 0

    mesh = plsc.VectorSubcoreMesh(core_axis_name="c", subcore_axis_name="s")

    def body(hs_h, src_h, dst_h, out_h, acc, is0, is1, id0, id1,
             rows0, rows1, tidx_s, tidx_d, trows,
             sem_s0, sem_s1, sem_d0, sem_d1, sem_g0, sem_g1, sem_t):
        c = lax.axis_index("c")
        s = lax.axis_index("s")
        e0 = (c * _NSUB + s) * epc
        r0 = s * rpc
        idx_s = (is0, is1)
        idx_d = (id0, id1)
        rows = (rows0, rows1)
        sem_s = (sem_s0, sem_s1)
        sem_d = (sem_d0, sem_d1)
        sem_g = (sem_g0, sem_g1)

        def start_idx(t, b):
            tt = jnp.minimum(t, nt - 1)
            pltpu.async_copy(src_h.at[pl.ds(e0 + tt * _ET, _ET)],
                             idx_s[b], sem_s[b])
            pltpu.async_copy(dst_h.at[pl.ds(e0 + tt * _ET, _ET)],
                             idx_d[b], sem_d[b])

        def wait_idx(b):
            pltpu.make_async_copy(src_h.at[pl.ds(0, _ET)],
                                  idx_s[b], sem_s[b]).wait()
            pltpu.make_async_copy(dst_h.at[pl.ds(0, _ET)],
                                  idx_d[b], sem_d[b]).wait()

        def start_gather(b):
            pltpu.async_copy(hs_h.at[idx_s[b]], rows[b], sem_g[b])

        def wait_gather(b):
            pltpu.make_async_copy(hs_h.at[pl.ds(0, _ET)],
                                  rows[b], sem_g[b]).wait()

        pltpu.sync_copy(hs_h.at[pl.ds(r0, rpc)], acc.at[pl.ds(r0, rpc)])
        plsc.subcore_barrier()

        start_idx(0, 0)
        start_idx(1, 1)
        wait_idx(0)
        start_gather(0)

        @pl.loop(0, nt, step=2)
        def _(g):
            for b in (0, 1):
                t = g + b
                nb = 1 - b
                wait_gather(b)
                wait_idx(nb)

                @pl.when(t + 1 < nt)
                def _():
                    start_gather(nb)

                pltpu.sync_copy(rows[b], acc.at[idx_d[b]], add=True)
                start_idx(t + 2, b)

        wait_idx(1)

        if tail:
            pltpu.sync_copy(src_h.at[pl.ds(e0 + nt * _ET, tail)], tidx_s)
            pltpu.sync_copy(dst_h.at[pl.ds(e0 + nt * _ET, tail)], tidx_d)
            pltpu.async_copy(hs_h.at[tidx_s], trows, sem_t).wait()
            pltpu.sync_copy(trows, acc.at[tidx_d], add=True)

        plsc.subcore_barrier()
        pltpu.sync_copy(acc.at[pl.ds(r0, rpc)], out_h.at[c, pl.ds(r0, rpc)])

    f = pl.kernel(
        body,
        out_type=jax.ShapeDtypeStruct((2, npad, fw), jnp.float32),
        mesh=mesh,
        scratch_types=[
            pltpu.VMEM_SHARED((npad, fw), jnp.float32),
            pltpu.VMEM((_ET,), jnp.int32),
            pltpu.VMEM((_ET,), jnp.int32),
            pltpu.VMEM((_ET,), jnp.int32),
            pltpu.VMEM((_ET,), jnp.int32),
            pltpu.VMEM((_ET, fw), jnp.float32),
            pltpu.VMEM((_ET, fw), jnp.float32),
            pltpu.VMEM((max(tail, 8),), jnp.int32),
            pltpu.VMEM((max(tail, 8),), jnp.int32),
            pltpu.VMEM((max(tail, 8), fw), jnp.float32),
            pltpu.SemaphoreType.DMA,
            pltpu.SemaphoreType.DMA,
            pltpu.SemaphoreType.DMA,
            pltpu.SemaphoreType.DMA,
            pltpu.SemaphoreType.DMA,
            pltpu.SemaphoreType.DMA,
            pltpu.SemaphoreType.DMA,
        ],
    )
    out = f(hs, src, dst)
    return out[0, :n] + out[1, :n] - hs[:n]


def _sc_deg_call(dst, n):
    e = dst.shape[0]
    npad = _sc_pad(n)
    fw = 128
    dt = 80
    epc = e // (2 * _NSUB)
    rpc = npad // _NSUB
    assert epc * 2 * _NSUB == e and epc % dt == 0
    ones_n = jnp.ones((npad, fw), jnp.float32)

    mesh = plsc.VectorSubcoreMesh(core_axis_name="c", subcore_axis_name="s")

    def body(dst_h, ones_h, out_h, acc, ones_v, idx_d, sem):
        c = lax.axis_index("c")
        s = lax.axis_index("s")
        e0 = (c * _NSUB + s) * epc
        r0 = s * rpc

        pltpu.sync_copy(ones_h.at[pl.ds(r0, rpc)], acc.at[pl.ds(r0, rpc)])
        pltpu.sync_copy(ones_h.at[pl.ds(0, dt)], ones_v)
        plsc.subcore_barrier()

        @pl.loop(0, epc, step=dt)
        def _(t):
            pltpu.sync_copy(dst_h.at[pl.ds(e0 + t, dt)], idx_d)
            pltpu.sync_copy(ones_v, acc.at[idx_d], add=True)

        plsc.subcore_barrier()
        pltpu.sync_copy(acc.at[pl.ds(r0, rpc)], out_h.at[c, pl.ds(r0, rpc)])

    f = pl.kernel(
        body,
        out_type=jax.ShapeDtypeStruct((2, npad, fw), jnp.float32),
        mesh=mesh,
        scratch_types=[
            pltpu.VMEM_SHARED((npad, fw), jnp.float32),
            pltpu.VMEM((dt, fw), jnp.float32),
            pltpu.VMEM((dt,), jnp.int32),
            pltpu.SemaphoreType.DMA,
        ],
    )
    out = f(dst, ones_n)
    return out[0, :n, 0] + out[1, :n, 0] - 1.0


def _mm_bias_kernel(x_ref, w_ref, b_ref, o_ref):
    o_ref[...] = (
        jnp.dot(x_ref[...], w_ref[...], preferred_element_type=jnp.float32)
        + b_ref[...]
    )


def _mm(x, w, b, block=1000):
    n, k = x.shape
    f = w.shape[1]
    b2 = b.reshape(1, f)
    return pl.pallas_call(
        _mm_bias_kernel,
        grid=(n // block,),
        in_specs=[
            pl.BlockSpec((block, k), lambda i: (i, 0)),
            pl.BlockSpec((k, f), lambda i: (0, 0)),
            pl.BlockSpec((1, f), lambda i: (0, 0)),
        ],
        out_specs=pl.BlockSpec((block, f), lambda i: (i, 0)),
        out_shape=jax.ShapeDtypeStruct((n, f), jnp.float32),
    )(x, w, b2)


def _bn(h, g, b, eps=1e-5):
    m = jnp.mean(h, axis=0)
    v = jnp.var(h, axis=0)
    return g * (h - m) * jax.lax.rsqrt(v + eps) + b


def kernel(x, edge_index, batch, params):
    n = x.shape[0]
    src, dst = edge_index[0], edge_index[1]

    deg = _sc_deg_call(dst, n)
    dinv = jax.lax.rsqrt(deg)

    def gcn(h, W, b):
        hs = (h @ W) * dinv[:, None]
        fh = hs.shape[1]
        if fh == 128:
            agg = _sc_agg_call(hs, src, dst)
        else:
            aggs = [_sc_agg_call(hs[:, i:i + 128], src, dst)
                    for i in range(0, fh, 128)]
            agg = jnp.concatenate(aggs, axis=1)
        return dinv[:, None] * agg + b

    def block(h, p):
        res = h @ p['Ws'] + p['bs']
        u = jax.nn.relu(_bn(gcn(h, p['W1'], p['b1']), p['g1'], p['bb1']))
        u = _bn(gcn(u, p['W2'], p['b2']), p['g2'], p['bb2'])
        return jax.nn.relu(u + res)

    h = block(x, params['blk1'])
    h = block(h, params['blk2'])
    nrm = jnp.maximum(jnp.linalg.norm(h, axis=1, keepdims=True), 1e-12)
    z = h / nrm

    zw = z @ params['We']
    edge_probs = jax.nn.sigmoid(jnp.sum(zw[src] * z[dst], axis=-1))

    t = jax.nn.relu(_mm(z, params['Wf1'], params['bf1']))
    hdec = _bn(t, params['gf'], params['bbf'])
    x_recon = _mm(hdec, params['Wf2'], params['bf2'])

    z_g = jax.ops.segment_max(z, batch, num_segments=100)
    z_g_mlp = jax.nn.relu(z_g @ params['Wp1'] + params['bp1']) @ params['Wp2'] + params['bp2']
    return (z, x_recon, edge_probs, z_g, z_g_mlp)
